# baseline (speedup 1.0000x reference)
import jax
import jax.numpy as jnp
from jax.experimental import pallas as pl

N_CLASSES = 1000
N_CTX = 8
NAME_LEN = 77
OUT_LEN = N_CTX + NAME_LEN
CTX_DIM = 512

BLOCK_C = 8


def _concat_body(ctx_ref, name_ref, out_ref):
    out_ref[:, 0:N_CTX, :] = ctx_ref[...]
    out_ref[:, N_CTX:OUT_LEN, :] = name_ref[...]


def kernel(ctx, name_embeds):
    grid = (N_CLASSES // BLOCK_C,)
    return pl.pallas_call(
        _concat_body,
        grid=grid,
        in_specs=[
            pl.BlockSpec((BLOCK_C, N_CTX, CTX_DIM), lambda i: (i, 0, 0)),
            pl.BlockSpec((BLOCK_C, NAME_LEN, CTX_DIM), lambda i: (i, 0, 0)),
        ],
        out_specs=pl.BlockSpec((BLOCK_C, OUT_LEN, CTX_DIM), lambda i: (i, 0, 0)),
        out_shape=jax.ShapeDtypeStruct((N_CLASSES, OUT_LEN, CTX_DIM), jnp.float32),
    )(ctx, name_embeds)


# speedup vs baseline: 1.0907x; 1.0907x over previous
import jax
import jax.numpy as jnp
from jax.experimental import pallas as pl

N_CLASSES = 1000
N_CTX = 8
NAME_LEN = 77
OUT_LEN = N_CTX + NAME_LEN
CTX_DIM = 512

BLOCK_C = 50


def _concat_body(ctx_ref, name_ref, out_ref):
    out_ref[:, 0:N_CTX, :] = ctx_ref[...]
    out_ref[:, N_CTX:OUT_LEN, :] = name_ref[...]


def kernel(ctx, name_embeds):
    grid = (N_CLASSES // BLOCK_C,)
    return pl.pallas_call(
        _concat_body,
        grid=grid,
        in_specs=[
            pl.BlockSpec((BLOCK_C, N_CTX, CTX_DIM), lambda i: (i, 0, 0)),
            pl.BlockSpec((BLOCK_C, NAME_LEN, CTX_DIM), lambda i: (i, 0, 0)),
        ],
        out_specs=pl.BlockSpec((BLOCK_C, OUT_LEN, CTX_DIM), lambda i: (i, 0, 0)),
        out_shape=jax.ShapeDtypeStruct((N_CLASSES, OUT_LEN, CTX_DIM), jnp.float32),
    )(ctx, name_embeds)
